# Initial kernel scaffold; baseline (speedup 1.0000x reference)
import jax
import jax.numpy as jnp
from jax.experimental import pallas as pl


def kernel(X, A_tilde, epsilon, W1, W_mu, W_logsigma):
    raise NotImplementedError("write your pallas kernel here")


# speedup vs baseline: 1.2115x; 1.2115x over previous
import jax
import jax.numpy as jnp
from jax.experimental import pallas as pl
from jax.experimental.pallas import tpu as pltpu

_BLK = 400


def _xw_kernel(x_ref, w_ref, o_ref):
    o_ref[...] = jnp.dot(x_ref[...], w_ref[...],
                         preferred_element_type=jnp.float32)


def _layer1_kernel(a_ref, xw_ref, wm_ref, wl_ref, hwm_ref, hwl_ref):
    h = jnp.maximum(
        jnp.dot(a_ref[...], xw_ref[...], preferred_element_type=jnp.float32),
        0.0)
    hwm_ref[...] = jnp.dot(h, wm_ref[...], preferred_element_type=jnp.float32)
    hwl_ref[...] = jnp.dot(h, wl_ref[...], preferred_element_type=jnp.float32)


def _layer2_kernel(a_ref, hwm_ref, hwl_ref, eps_ref, mu_ref, ls_ref, z_ref):
    a = a_ref[...]
    mu = jnp.dot(a, hwm_ref[...], preferred_element_type=jnp.float32)
    ls = jnp.dot(a, hwl_ref[...], preferred_element_type=jnp.float32)
    mu_ref[...] = mu
    ls_ref[...] = ls
    z_ref[...] = mu + eps_ref[...] * jnp.exp(ls)


def _decoder_kernel(z_ref, zt_ref, o_ref):
    logits = jnp.dot(z_ref[...], zt_ref[...],
                     preferred_element_type=jnp.float32)
    o_ref[...] = jax.nn.sigmoid(logits)


def kernel(X, A_tilde, epsilon, W1, W_mu, W_logsigma):
    n, f_in = X.shape
    hid = W1.shape[1]
    lat = W_mu.shape[1]
    blk = _BLK
    grid = (n // blk,)
    params = pltpu.CompilerParams(dimension_semantics=("parallel",))

    xw = pl.pallas_call(
        _xw_kernel,
        out_shape=jax.ShapeDtypeStruct((n, hid), jnp.float32),
    )(X, W1)

    hwm, hwl = pl.pallas_call(
        _layer1_kernel,
        grid=grid,
        in_specs=[
            pl.BlockSpec((blk, n), lambda i: (i, 0)),
            pl.BlockSpec((n, hid), lambda i: (0, 0)),
            pl.BlockSpec((hid, lat), lambda i: (0, 0)),
            pl.BlockSpec((hid, lat), lambda i: (0, 0)),
        ],
        out_specs=[
            pl.BlockSpec((blk, lat), lambda i: (i, 0)),
            pl.BlockSpec((blk, lat), lambda i: (i, 0)),
        ],
        out_shape=[
            jax.ShapeDtypeStruct((n, lat), jnp.float32),
            jax.ShapeDtypeStruct((n, lat), jnp.float32),
        ],
        compiler_params=params,
    )(A_tilde, xw, W_mu, W_logsigma)

    mu, logsigma, z = pl.pallas_call(
        _layer2_kernel,
        grid=grid,
        in_specs=[
            pl.BlockSpec((blk, n), lambda i: (i, 0)),
            pl.BlockSpec((n, lat), lambda i: (0, 0)),
            pl.BlockSpec((n, lat), lambda i: (0, 0)),
            pl.BlockSpec((blk, lat), lambda i: (i, 0)),
        ],
        out_specs=[
            pl.BlockSpec((blk, lat), lambda i: (i, 0)),
            pl.BlockSpec((blk, lat), lambda i: (i, 0)),
            pl.BlockSpec((blk, lat), lambda i: (i, 0)),
        ],
        out_shape=[
            jax.ShapeDtypeStruct((n, lat), jnp.float32),
            jax.ShapeDtypeStruct((n, lat), jnp.float32),
            jax.ShapeDtypeStruct((n, lat), jnp.float32),
        ],
        compiler_params=params,
    )(A_tilde, hwm, hwl, epsilon)

    a_hat = pl.pallas_call(
        _decoder_kernel,
        grid=grid,
        in_specs=[
            pl.BlockSpec((blk, lat), lambda i: (i, 0)),
            pl.BlockSpec((lat, n), lambda i: (0, 0)),
        ],
        out_specs=pl.BlockSpec((blk, n), lambda i: (i, 0)),
        out_shape=jax.ShapeDtypeStruct((n, n), jnp.float32),
        compiler_params=params,
    )(z, z.T)

    return (a_hat, mu, logsigma)
